# baseline (speedup 1.0000x reference)
import functools

import jax
import jax.numpy as jnp
from jax import lax
from jax.experimental import pallas as pl
from jax.experimental.pallas import tpu as pltpu
from jax.experimental.pallas import tpu_sc as plsc

VOCAB = 1000000
EMB_DIM = 64
BATCH = 4096
HIST = 50

_B = BATCH * HIST
_NC = 2
_NS = 16
_NW = _NC * _NS
_BPW = _B // _NW
_CH = 800
_NCHUNK = _BPW // _CH


def _gather_kernel(x_hbm, table_hbm, out_hbm, idx_v, buf0, buf1, sem0, sem1):
    wid = lax.axis_index("s") * _NC + lax.axis_index("c")
    base = wid * _BPW
    pltpu.sync_copy(x_hbm.at[pl.ds(base, _BPW)], idx_v)

    bufs = (buf0, buf1)
    sems = (sem0, sem1)

    cp0 = pltpu.async_copy(table_hbm.at[idx_v.at[pl.ds(0, _CH)]], buf0, sem0)

    def body(c, _):
        slot = lax.rem(c, 2)
        nxt = c + 1

        @pl.when(nxt < _NCHUNK)
        def _start():
            def start_into(s):
                pltpu.async_copy(
                    table_hbm.at[idx_v.at[pl.ds(nxt * _CH, _CH)]],
                    bufs[s], sems[s])
            lax.cond(lax.rem(nxt, 2) == 0,
                     lambda: start_into(0), lambda: start_into(1))

        def drain(s):
            pltpu.make_async_copy(
                table_hbm.at[idx_v.at[pl.ds(0, _CH)]], bufs[s], sems[s]).wait()
            pltpu.sync_copy(bufs[s], out_hbm.at[pl.ds(base + c * _CH, _CH)])
        lax.cond(slot == 0, lambda: drain(0), lambda: drain(1))
        return _

    lax.fori_loop(0, _NCHUNK, body, 0)


@jax.jit
def _embed(x_flat, table):
    mesh = plsc.VectorSubcoreMesh(core_axis_name="c", subcore_axis_name="s")
    f = functools.partial(
        pl.kernel,
        mesh=mesh,
        out_type=jax.ShapeDtypeStruct((_B, EMB_DIM), jnp.float32),
        scratch_types=[
            pltpu.VMEM((_BPW,), jnp.int32),
            pltpu.VMEM((_CH, EMB_DIM), jnp.float32),
            pltpu.VMEM((_CH, EMB_DIM), jnp.float32),
            pltpu.SemaphoreType.DMA,
            pltpu.SemaphoreType.DMA,
        ],
        compiler_params=pltpu.CompilerParams(use_tc_tiling_on_sc=False),
    )(_gather_kernel)
    return f(x_flat, table)


def kernel(x, table):
    out = _embed(x.reshape(_B), table)
    return out.reshape(BATCH, HIST, EMB_DIM)


# speedup vs baseline: 1.0001x; 1.0001x over previous
import functools

import jax
import jax.numpy as jnp
from jax import lax
from jax.experimental import pallas as pl
from jax.experimental.pallas import tpu as pltpu
from jax.experimental.pallas import tpu_sc as plsc

VOCAB = 1000000
EMB_DIM = 64
BATCH = 4096
HIST = 50

_B = BATCH * HIST
_NC = 2
_NS = 16
_NW = _NC * _NS
_BPW = _B // _NW
_NBUF = 4
_CH = 400
_NCHUNK = _BPW // _CH


def _gather_kernel(x_hbm, table_hbm, out_hbm, idx_v, bufs, sems):
    wid = lax.axis_index("s") * _NC + lax.axis_index("c")
    base = wid * _BPW
    pltpu.sync_copy(x_hbm.at[pl.ds(base, _BPW)], idx_v)

    def fire(c):
        s = c % _NBUF
        pltpu.async_copy(
            table_hbm.at[idx_v.at[pl.ds(c * _CH, _CH)]], bufs[s], sems[s])

    for c in range(_NBUF):
        fire(c)
    for c in range(_NCHUNK):
        s = c % _NBUF
        pltpu.make_async_copy(
            table_hbm.at[idx_v.at[pl.ds(c * _CH, _CH)]], bufs[s], sems[s]
        ).wait()
        pltpu.sync_copy(bufs[s], out_hbm.at[pl.ds(base + c * _CH, _CH)])
        if c + _NBUF < _NCHUNK:
            fire(c + _NBUF)


@jax.jit
def _embed(x_flat, table):
    mesh = plsc.VectorSubcoreMesh(core_axis_name="c", subcore_axis_name="s")
    f = functools.partial(
        pl.kernel,
        mesh=mesh,
        out_type=jax.ShapeDtypeStruct((_B, EMB_DIM), jnp.float32),
        scratch_types=[
            pltpu.VMEM((_BPW,), jnp.int32),
            [pltpu.VMEM((_CH, EMB_DIM), jnp.float32) for _ in range(_NBUF)],
            [pltpu.SemaphoreType.DMA for _ in range(_NBUF)],
        ],
        compiler_params=pltpu.CompilerParams(use_tc_tiling_on_sc=False),
    )(_gather_kernel)
    return f(x_flat, table)


def kernel(x, table):
    out = _embed(x.reshape(_B), table)
    return out.reshape(BATCH, HIST, EMB_DIM)
